# baseline (speedup 1.0000x reference)
import jax
import jax.numpy as jnp
from jax import lax
from jax.experimental import pallas as pl
from jax.experimental.pallas import tpu as pltpu
from jax.experimental.pallas import tpu_sc as plsc

_K = 9
_OUT_ELEMS = 2 * 8 * 10000 * _K
_NW = 32
_PER_W = _OUT_ELEMS // _NW
_CH = 7680
_NCH = -(-_PER_W // _CH)
_LAST = _PER_W - _CH
_UNROLL = 8


def _deinterleave_body(in_hbm, out_hbm, in_v, out_v):
    c = lax.axis_index("c")
    s = lax.axis_index("s")
    wid = s * 2 + c
    base = wid * _PER_W
    iota2 = lax.iota(jnp.int32, 16) * 2

    def chunk(i, carry):
        start = base + jnp.minimum(i * _CH, _LAST)
        pltpu.sync_copy(in_hbm.at[pl.ds(start * 2, _CH * 2)], in_v)

        def vec(j, inner):
            for u in range(_UNROLL):
                v = j * _UNROLL + u
                out_v[pl.ds(v * 16, 16)] = plsc.load_gather(
                    in_v, [iota2 + v * 32])
            return inner

        lax.fori_loop(0, _CH // 16 // _UNROLL, vec, 0)
        pltpu.sync_copy(out_v, out_hbm.at[pl.ds(start, _CH)])
        return carry

    lax.fori_loop(0, _NCH, chunk, 0)


def kernel(edge_index):
    flat = edge_index.reshape(-1)
    out = pl.kernel(
        _deinterleave_body,
        out_type=jax.ShapeDtypeStruct((_OUT_ELEMS,), jnp.int32),
        mesh=plsc.VectorSubcoreMesh(core_axis_name="c", subcore_axis_name="s"),
        compiler_params=pltpu.CompilerParams(needs_layout_passes=False),
        scratch_types=[
            pltpu.VMEM((_CH * 2,), jnp.int32),
            pltpu.VMEM((_CH,), jnp.int32),
        ],
    )(flat)
    sh = edge_index.shape
    return out.reshape(sh[0], sh[1], sh[2], _K)
